# baseline (speedup 1.0000x reference)
import jax
import jax.numpy as jnp
from jax import lax
from jax.experimental import pallas as pl


def _matmul1_tc(x2d, W, dis_col, nb, db):
    def body(x_ref, w_ref, d_ref, o_ref):
        o_ref[...] = jnp.dot(x_ref[...], w_ref[...],
                             preferred_element_type=jnp.float32) * d_ref[...]

    R = x2d.shape[0]
    return pl.pallas_call(
        body,
        grid=(R // nb,),
        in_specs=[
            pl.BlockSpec((nb, 128), lambda i: (i, 0)),
            pl.BlockSpec((128, 128), lambda i: (0, 0)),
            pl.BlockSpec((nb, 1), lambda i: (i % db, 0)),
        ],
        out_specs=pl.BlockSpec((nb, 128), lambda i: (i, 0)),
        out_shape=jax.ShapeDtypeStruct((R, 128), jnp.float32),
    )(x2d, W, dis_col)


def _matmul2_tc(s1, W, bias, dis_col, nb, db):
    def body(a_ref, w_ref, bi_ref, d_ref, o_ref):
        x = jax.nn.relu(a_ref[...] * d_ref[...] + bi_ref[...])
        o_ref[...] = jnp.dot(x, w_ref[...],
                             preferred_element_type=jnp.float32) * d_ref[...]

    R = s1.shape[0]
    return pl.pallas_call(
        body,
        grid=(R // nb,),
        in_specs=[
            pl.BlockSpec((nb, 128), lambda i: (i, 0)),
            pl.BlockSpec((128, 128), lambda i: (0, 0)),
            pl.BlockSpec((1, 128), lambda i: (0, 0)),
            pl.BlockSpec((nb, 1), lambda i: (i % db, 0)),
        ],
        out_specs=pl.BlockSpec((nb, 128), lambda i: (i, 0)),
        out_shape=jax.ShapeDtypeStruct((R, 128), jnp.float32),
    )(s1, W, bias, dis_col)


def _final_tc(s2, bias, dis_col, nb, db):
    def body(a_ref, bi_ref, d_ref, o_ref):
        o_ref[...] = jnp.tanh(a_ref[...] * d_ref[...] + bi_ref[...])

    R = s2.shape[0]
    return pl.pallas_call(
        body,
        grid=(R // nb,),
        in_specs=[
            pl.BlockSpec((nb, 128), lambda i: (i, 0)),
            pl.BlockSpec((1, 128), lambda i: (0, 0)),
            pl.BlockSpec((nb, 1), lambda i: (i % db, 0)),
        ],
        out_specs=pl.BlockSpec((nb, 128), lambda i: (i, 0)),
        out_shape=jax.ShapeDtypeStruct((R, 128), jnp.float32),
    )(s2, bias, dis_col)


def kernel(x, mask, spatial_edge_index, spatial_edge_weight, W1, b1, W2, b2):
    del mask
    T, N, F = x.shape
    E = spatial_edge_weight.shape[0]
    NP = ((N + 1280 - 1) // 1280) * 1280
    row = spatial_edge_index[0]
    col = spatial_edge_index[1]
    ew = spatial_edge_weight

    deg = jnp.zeros((NP,), jnp.float32).at[col].add(ew)
    dis_col = jnp.where(deg > 0, lax.rsqrt(jnp.where(deg > 0, deg, 1.0)),
                        0.0)[:, None]

    nb = 1024
    db = NP // nb

    toff = (jnp.arange(T, dtype=jnp.int32) * NP)[:, None]
    rowf = (row[None, :] + toff).reshape(-1)
    colf = (col[None, :] + toff).reshape(-1)
    ewf = jnp.tile(ew, T)[:, None]

    def seg_sum(p):
        msg = p[rowf] * ewf
        return jnp.zeros_like(p).at[colf].add(msg)

    xp = jnp.pad(x, ((0, 0), (0, NP - N), (0, 0))).reshape(T * NP, F)
    p1 = _matmul1_tc(xp, W1, dis_col, nb, db)
    s1 = seg_sum(p1)
    p2 = _matmul2_tc(s1, W2, b1.reshape(1, 128), dis_col, nb, db)
    s2 = seg_sum(p2)
    out = _final_tc(s2, b2.reshape(1, 128), dis_col, nb, db)
    return out.reshape(T, NP, F)[:, :N, :]
